# Initial kernel scaffold; baseline (speedup 1.0000x reference)
import jax
import jax.numpy as jnp
from jax.experimental import pallas as pl


def kernel(user_idx, diner_idx, features, categorical_bucket_idx, user_table, diner_table, cat_tables, fn_g, fn_b, W1, b1, ln1_g, ln1_b, W2, b2, ln2_g, ln2_b, W3, b3):
    raise NotImplementedError("write your pallas kernel here")


# speedup vs baseline: 8.1153x; 8.1153x over previous
import functools

import jax
import jax.numpy as jnp
from jax.experimental import pallas as pl
from jax.experimental.pallas import tpu as pltpu
from jax.experimental.pallas import tpu_sc as plsc

B = 16384
ED = 16
NF = 13
NC, NB, CD = 26, 20, 8
NCLS = NC * NB
H1, H2 = 256, 128
PACK = 128 // ED

GATHER_W = 128
MLP_BLK = 512


def _sc_gather(user_packed, uidx, diner_packed, didx):
    mesh = plsc.VectorSubcoreMesh(core_axis_name="c", subcore_axis_name="s")

    @functools.partial(
        pl.kernel,
        out_type=(
            jax.ShapeDtypeStruct((B, 128), jnp.float32),
            jax.ShapeDtypeStruct((B, 128), jnp.float32),
        ),
        mesh=mesh,
    )
    def gather_kernel(ut_hbm, ui_hbm, dt_hbm, di_hbm, ue_hbm, de_hbm):
        def make_body(table_hbm):
            def body(i_vmem, o_vmem):
                pltpu.sync_copy(table_hbm.at[i_vmem.at[0]], o_vmem)
            return body

        def run(table_hbm, idx_hbm, out_hbm):
            pltpu.emit_pipeline(
                make_body(table_hbm),
                grid=(B // GATHER_W,),
                in_specs=[pl.BlockSpec((1, GATHER_W), lambda i: (0, i))],
                out_specs=[pl.BlockSpec((GATHER_W, 128), lambda i: (i, 0))],
                core_axis_name=("c", "s"),
                dimension_semantics=(pltpu.PARALLEL,),
            )(idx_hbm, out_hbm)

        run(ut_hbm, ui_hbm, ue_hbm)
        run(dt_hbm, di_hbm, de_hbm)

    return gather_kernel(user_packed, uidx, diner_packed, didx)


def _select_sub(packed, mod):
    acc = None
    for k in range(PACK):
        piece = packed[:, k * ED:(k + 1) * ED] * (mod == float(k))
        acc = piece if acc is None else acc + piece
    return acc


def _mlp_body(uep, umod, dep, dmod, f, bkt, expand, patt, Wcat, W1r, b1,
              fn_g, fn_b, g1, bb1, W2, b2, g2, bb2, W3, b3, out):
    ue = _select_sub(uep[...], umod[...])
    de = _select_sub(dep[...], dmod[...])

    fx = f[...]
    m = jnp.mean(fx, axis=-1, keepdims=True)
    v = jnp.mean((fx - m) ** 2, axis=-1, keepdims=True)
    fln = (fx - m) * jax.lax.rsqrt(v + 1e-5) * fn_g[...] + fn_b[...]

    rep = jnp.dot(bkt[...], expand[...],
                  preferred_element_type=jnp.float32)
    mh = (rep == patt[...]).astype(jnp.bfloat16)
    h = jnp.dot(mh, Wcat[...], preferred_element_type=jnp.float32)

    xr = jnp.concatenate([ue, de, fln], axis=-1).astype(jnp.bfloat16)
    h = h + jnp.dot(xr, W1r[...], preferred_element_type=jnp.float32)
    h = h + b1[...]
    m = jnp.mean(h, axis=-1, keepdims=True)
    v = jnp.mean((h - m) ** 2, axis=-1, keepdims=True)
    h = (h - m) * jax.lax.rsqrt(v + 1e-5) * g1[...] + bb1[...]
    h = jnp.maximum(h, 0.0).astype(jnp.bfloat16)

    h = jnp.dot(h, W2[...], preferred_element_type=jnp.float32) + b2[...]
    m = jnp.mean(h, axis=-1, keepdims=True)
    v = jnp.mean((h - m) ** 2, axis=-1, keepdims=True)
    h = (h - m) * jax.lax.rsqrt(v + 1e-5) * g2[...] + bb2[...]
    h = jnp.maximum(h, 0.0).astype(jnp.bfloat16)

    o = jnp.dot(h, W3[...], preferred_element_type=jnp.float32) + b3[...]
    out[...] = jax.nn.sigmoid(o)


def _tc_mlp(uep, umod, dep, dmod, features, bkt, expand, patt, Wcat, W1r,
            b1, fn_g, fn_b, ln1_g, ln1_b, W2, b2, ln2_g, ln2_b, W3, b3):
    grid = (B // MLP_BLK,)

    def row_spec(cols):
        return pl.BlockSpec((MLP_BLK, cols), lambda i: (i, 0))

    def full_spec(a):
        return pl.BlockSpec(a.shape, lambda i: (0,) * a.ndim)

    out = pl.pallas_call(
        _mlp_body,
        grid=grid,
        in_specs=[
            row_spec(128), row_spec(1), row_spec(128), row_spec(1),
            row_spec(NF), row_spec(NC),
            full_spec(expand), full_spec(patt), full_spec(Wcat),
            full_spec(W1r), full_spec(b1),
            full_spec(fn_g), full_spec(fn_b),
            full_spec(ln1_g), full_spec(ln1_b),
            full_spec(W2), full_spec(b2), full_spec(ln2_g), full_spec(ln2_b),
            full_spec(W3), full_spec(b3),
        ],
        out_specs=pl.BlockSpec((MLP_BLK, 1), lambda i: (i, 0)),
        out_shape=jax.ShapeDtypeStruct((B, 1), jnp.float32),
    )(uep, umod, dep, dmod, features, bkt, expand, patt, Wcat, W1r, b1,
      fn_g, fn_b, ln1_g, ln1_b, W2, b2, ln2_g, ln2_b, W3, b3)
    return out[:, 0]


def kernel(user_idx, diner_idx, features, categorical_bucket_idx,
           user_table, diner_table, cat_tables, fn_g, fn_b, W1, b1,
           ln1_g, ln1_b, W2, b2, ln2_g, ln2_b, W3, b3):
    uidx = user_idx.astype(jnp.int32)
    didx = diner_idx.astype(jnp.int32)

    user_packed = user_table.reshape(-1, 128)
    diner_packed = diner_table.reshape(-1, 128)

    uep, dep = _sc_gather(user_packed, (uidx // PACK).reshape(1, B),
                          diner_packed, (didx // PACK).reshape(1, B))
    umod = (uidx % PACK).astype(jnp.float32).reshape(B, 1)
    dmod = (didx % PACK).astype(jnp.float32).reshape(B, 1)

    W1c = W1[2 * ED + NF:].reshape(NC, CD, H1)
    Wcat = jnp.einsum("cbd,cdh->cbh", cat_tables, W1c,
                      preferred_element_type=jnp.float32)
    Wcat = Wcat.reshape(NCLS, H1).astype(jnp.bfloat16)

    cls = jnp.arange(NCLS, dtype=jnp.int32)
    expand = (cls[None, :] // NB == jnp.arange(NC, dtype=jnp.int32)[:, None])
    expand = expand.astype(jnp.bfloat16)
    patt = (cls % NB).astype(jnp.float32).reshape(1, NCLS)
    bkt = categorical_bucket_idx.astype(jnp.bfloat16)

    out = _tc_mlp(uep, umod, dep, dmod, features, bkt, expand, patt, Wcat,
                  W1[:2 * ED + NF].astype(jnp.bfloat16),
                  b1.reshape(1, H1),
                  fn_g.reshape(1, NF), fn_b.reshape(1, NF),
                  ln1_g.reshape(1, H1), ln1_b.reshape(1, H1),
                  W2.astype(jnp.bfloat16), b2.reshape(1, H2),
                  ln2_g.reshape(1, H2), ln2_b.reshape(1, H2),
                  W3.astype(jnp.bfloat16), b3.reshape(1, 1))
    return out


# speedup vs baseline: 9.8065x; 1.2084x over previous
import functools

import jax
import jax.numpy as jnp
from jax.experimental import pallas as pl
from jax.experimental.pallas import tpu as pltpu
from jax.experimental.pallas import tpu_sc as plsc

B = 16384
ED = 16
NF = 13
NC, NB, CD = 26, 20, 8
NCLS = NC * NB
H1, H2 = 256, 128
PACK = 128 // ED

GATHER_W = 128
MLP_BLK = 512


def _sc_gather(user_table, uidx, diner_table, didx):
    mesh = plsc.VectorSubcoreMesh(core_axis_name="c", subcore_axis_name="s")

    @functools.partial(
        pl.kernel,
        out_type=(
            jax.ShapeDtypeStruct((B, 128), jnp.float32),
            jax.ShapeDtypeStruct((B, 128), jnp.float32),
        ),
        mesh=mesh,
    )
    def gather_kernel(ut_hbm, ui_hbm, dt_hbm, di_hbm, ue_hbm, de_hbm):
        def make_body(table_hbm):
            def body(i_vmem, o_vmem):
                pltpu.sync_copy(table_hbm.at[i_vmem.at[0]], o_vmem)
            return body

        def run(table_hbm, idx_hbm, out_hbm):
            pltpu.emit_pipeline(
                make_body(table_hbm),
                grid=(B // GATHER_W,),
                in_specs=[pl.BlockSpec((1, GATHER_W), lambda i: (0, i))],
                out_specs=[pl.BlockSpec((GATHER_W, 128), lambda i: (i, 0))],
                core_axis_name=("c", "s"),
                dimension_semantics=(pltpu.PARALLEL,),
            )(idx_hbm, out_hbm)

        run(ut_hbm, ui_hbm, ue_hbm)
        run(dt_hbm, di_hbm, de_hbm)

    return gather_kernel(user_table, uidx, diner_table, didx)


def _mlp_body(uep, uidx, dep, didx, f, bkt, kpat, expand, patt, Wcat,
              Wu, Wd, Wf, b1, fn_g, fn_b, g1, bb1, W2, b2, g2, bb2,
              W3, b3, out):
    mu = ((uidx[...] % PACK) == kpat[...]).astype(jnp.bfloat16)
    md = ((didx[...] % PACK) == kpat[...]).astype(jnp.bfloat16)
    pu = uep[...].astype(jnp.bfloat16) * mu
    pd = dep[...].astype(jnp.bfloat16) * md

    fx = f[...]
    m = jnp.mean(fx, axis=-1, keepdims=True)
    v = jnp.mean((fx - m) ** 2, axis=-1, keepdims=True)
    fln = (fx - m) * jax.lax.rsqrt(v + 1e-5) * fn_g[...] + fn_b[...]

    rep = jnp.dot(bkt[...], expand[...], preferred_element_type=jnp.float32)
    mh = (rep == patt[...]).astype(jnp.bfloat16)

    h = jnp.dot(mh, Wcat[...], preferred_element_type=jnp.float32)
    h = h + jnp.dot(pu, Wu[...], preferred_element_type=jnp.float32)
    h = h + jnp.dot(pd, Wd[...], preferred_element_type=jnp.float32)
    h = h + jnp.dot(fln.astype(jnp.bfloat16), Wf[...],
                    preferred_element_type=jnp.float32)
    h = h + b1[...]
    m = jnp.mean(h, axis=-1, keepdims=True)
    v = jnp.mean((h - m) ** 2, axis=-1, keepdims=True)
    h = (h - m) * jax.lax.rsqrt(v + 1e-5) * g1[...] + bb1[...]
    h = jnp.maximum(h, 0.0).astype(jnp.bfloat16)

    h = jnp.dot(h, W2[...], preferred_element_type=jnp.float32) + b2[...]
    m = jnp.mean(h, axis=-1, keepdims=True)
    v = jnp.mean((h - m) ** 2, axis=-1, keepdims=True)
    h = (h - m) * jax.lax.rsqrt(v + 1e-5) * g2[...] + bb2[...]
    h = jnp.maximum(h, 0.0).astype(jnp.bfloat16)

    o = jnp.dot(h, W3[...], preferred_element_type=jnp.float32) + b3[...]
    out[...] = jax.nn.sigmoid(o)


def _tc_mlp(uep, uidx, dep, didx, features, bkt, kpat, expand, patt, Wcat,
            Wu, Wd, Wf, b1, fn_g, fn_b, ln1_g, ln1_b, W2, b2, ln2_g, ln2_b,
            W3, b3):
    grid = (B // MLP_BLK,)

    def row_spec(cols):
        return pl.BlockSpec((MLP_BLK, cols), lambda i: (i, 0))

    def full_spec(a):
        return pl.BlockSpec(a.shape, lambda i: (0,) * a.ndim)

    out = pl.pallas_call(
        _mlp_body,
        grid=grid,
        in_specs=[
            row_spec(128), row_spec(1), row_spec(128), row_spec(1),
            row_spec(NF), row_spec(NC),
            full_spec(kpat), full_spec(expand), full_spec(patt),
            full_spec(Wcat), full_spec(Wu), full_spec(Wd), full_spec(Wf),
            full_spec(b1), full_spec(fn_g), full_spec(fn_b),
            full_spec(ln1_g), full_spec(ln1_b),
            full_spec(W2), full_spec(b2), full_spec(ln2_g), full_spec(ln2_b),
            full_spec(W3), full_spec(b3),
        ],
        out_specs=pl.BlockSpec((MLP_BLK, 1), lambda i: (i, 0)),
        out_shape=jax.ShapeDtypeStruct((B, 1), jnp.float32),
    )(uep, uidx, dep, didx, features, bkt, kpat, expand, patt, Wcat,
      Wu, Wd, Wf, b1, fn_g, fn_b, ln1_g, ln1_b, W2, b2, ln2_g, ln2_b,
      W3, b3)
    return out[:, 0]


def kernel(user_idx, diner_idx, features, categorical_bucket_idx,
           user_table, diner_table, cat_tables, fn_g, fn_b, W1, b1,
           ln1_g, ln1_b, W2, b2, ln2_g, ln2_b, W3, b3):
    uidx = user_idx.astype(jnp.int32)
    didx = diner_idx.astype(jnp.int32)

    uep, dep = _sc_gather(user_table.reshape(-1, 128),
                          (uidx // PACK).reshape(1, B),
                          diner_table.reshape(-1, 128),
                          (didx // PACK).reshape(1, B))

    W1c = W1[2 * ED + NF:].reshape(NC, CD, H1)
    Wcat = jnp.einsum("cbd,cdh->cbh", cat_tables, W1c,
                      preferred_element_type=jnp.float32)
    Wcat = Wcat.reshape(NCLS, H1).astype(jnp.bfloat16)

    cls = jnp.arange(NCLS, dtype=jnp.int32)
    expand = (cls[None, :] // NB == jnp.arange(NC, dtype=jnp.int32)[:, None])
    expand = expand.astype(jnp.bfloat16)
    patt = (cls % NB).astype(jnp.float32).reshape(1, NCLS)
    bkt = categorical_bucket_idx.astype(jnp.bfloat16)
    kpat = (jnp.arange(128, dtype=jnp.int32) // ED).reshape(1, 128)

    Wb = W1.astype(jnp.bfloat16)
    Wu = jnp.tile(Wb[:ED], (PACK, 1))
    Wd = jnp.tile(Wb[ED:2 * ED], (PACK, 1))
    Wf = Wb[2 * ED:2 * ED + NF]

    out = _tc_mlp(uep, uidx.reshape(B, 1), dep, didx.reshape(B, 1),
                  features, bkt, kpat, expand, patt, Wcat, Wu, Wd, Wf,
                  b1.reshape(1, H1),
                  fn_g.reshape(1, NF), fn_b.reshape(1, NF),
                  ln1_g.reshape(1, H1), ln1_b.reshape(1, H1),
                  W2.astype(jnp.bfloat16), b2.reshape(1, H2),
                  ln2_g.reshape(1, H2), ln2_b.reshape(1, H2),
                  W3.astype(jnp.bfloat16), b3.reshape(1, 1))
    return out


# speedup vs baseline: 12.3312x; 1.2575x over previous
import functools

import jax
import jax.numpy as jnp
from jax.experimental import pallas as pl
from jax.experimental.pallas import tpu as pltpu
from jax.experimental.pallas import tpu_sc as plsc

B = 16384
ED = 16
NF = 13
NC, NB, CD = 26, 20, 8
NCLS = NC * NB
H1, H2 = 256, 128
PACK = 128 // ED

GATHER_W = 128
MLP_BLK = 512


def _repack_body(in_ref, out_ref):
    x = in_ref[...]
    y = jnp.transpose(x)
    y3 = y.reshape(-1, PACK, ED)
    out_ref[...] = jnp.concatenate([y3[:, j, :] for j in range(PACK)],
                                   axis=1)


def _tc_repack(tT, col_block):
    n = tT.shape[1]
    grid = ((n + col_block - 1) // col_block,)
    return pl.pallas_call(
        _repack_body,
        grid=grid,
        in_specs=[pl.BlockSpec((ED, col_block), lambda i: (0, i))],
        out_specs=pl.BlockSpec((col_block // PACK, 128), lambda i: (i, 0)),
        out_shape=jax.ShapeDtypeStruct((n // PACK, 128), jnp.float32),
    )(tT)


def _sc_gather(user_table, uidx, diner_table, didx):
    mesh = plsc.VectorSubcoreMesh(core_axis_name="c", subcore_axis_name="s")

    @functools.partial(
        pl.kernel,
        out_type=(
            jax.ShapeDtypeStruct((B, 128), jnp.float32),
            jax.ShapeDtypeStruct((B, 128), jnp.float32),
        ),
        mesh=mesh,
    )
    def gather_kernel(ut_hbm, ui_hbm, dt_hbm, di_hbm, ue_hbm, de_hbm):
        def make_body(table_hbm):
            def body(i_vmem, o_vmem):
                pltpu.sync_copy(table_hbm.at[i_vmem.at[0]], o_vmem)
            return body

        def run(table_hbm, idx_hbm, out_hbm):
            pltpu.emit_pipeline(
                make_body(table_hbm),
                grid=(B // GATHER_W,),
                in_specs=[pl.BlockSpec((1, GATHER_W), lambda i: (0, i))],
                out_specs=[pl.BlockSpec((GATHER_W, 128), lambda i: (i, 0))],
                core_axis_name=("c", "s"),
                dimension_semantics=(pltpu.PARALLEL,),
            )(idx_hbm, out_hbm)

        run(ut_hbm, ui_hbm, ue_hbm)
        run(dt_hbm, di_hbm, de_hbm)

    return gather_kernel(user_table, uidx, diner_table, didx)


def _mlp_body(uep, uidx, dep, didx, f, bkt, kpat, expand, patt, Wcat,
              Wu, Wd, Wf, b1, fn_g, fn_b, g1, bb1, W2, b2, g2, bb2,
              W3, b3, out):
    mu = ((uidx[...] % PACK) == kpat[...]).astype(jnp.bfloat16)
    md = ((didx[...] % PACK) == kpat[...]).astype(jnp.bfloat16)
    pu = uep[...].astype(jnp.bfloat16) * mu
    pd = dep[...].astype(jnp.bfloat16) * md

    fx = f[...]
    m = jnp.mean(fx, axis=-1, keepdims=True)
    v = jnp.mean((fx - m) ** 2, axis=-1, keepdims=True)
    fln = (fx - m) * jax.lax.rsqrt(v + 1e-5) * fn_g[...] + fn_b[...]

    rep = jnp.dot(bkt[...], expand[...], preferred_element_type=jnp.float32)
    mh = (rep == patt[...]).astype(jnp.bfloat16)

    h = jnp.dot(mh, Wcat[...], preferred_element_type=jnp.float32)
    h = h + jnp.dot(pu, Wu[...], preferred_element_type=jnp.float32)
    h = h + jnp.dot(pd, Wd[...], preferred_element_type=jnp.float32)
    h = h + jnp.dot(fln.astype(jnp.bfloat16), Wf[...],
                    preferred_element_type=jnp.float32)
    h = h + b1[...]
    m = jnp.mean(h, axis=-1, keepdims=True)
    v = jnp.mean((h - m) ** 2, axis=-1, keepdims=True)
    h = (h - m) * jax.lax.rsqrt(v + 1e-5) * g1[...] + bb1[...]
    h = jnp.maximum(h, 0.0).astype(jnp.bfloat16)

    h = jnp.dot(h, W2[...], preferred_element_type=jnp.float32) + b2[...]
    m = jnp.mean(h, axis=-1, keepdims=True)
    v = jnp.mean((h - m) ** 2, axis=-1, keepdims=True)
    h = (h - m) * jax.lax.rsqrt(v + 1e-5) * g2[...] + bb2[...]
    h = jnp.maximum(h, 0.0).astype(jnp.bfloat16)

    o = jnp.dot(h, W3[...], preferred_element_type=jnp.float32) + b3[...]
    out[...] = jax.nn.sigmoid(o)


def _tc_mlp(uep, uidx, dep, didx, features, bkt, kpat, expand, patt, Wcat,
            Wu, Wd, Wf, b1, fn_g, fn_b, ln1_g, ln1_b, W2, b2, ln2_g, ln2_b,
            W3, b3):
    grid = (B // MLP_BLK,)

    def row_spec(cols):
        return pl.BlockSpec((MLP_BLK, cols), lambda i: (i, 0))

    def full_spec(a):
        return pl.BlockSpec(a.shape, lambda i: (0,) * a.ndim)

    out = pl.pallas_call(
        _mlp_body,
        grid=grid,
        in_specs=[
            row_spec(128), row_spec(1), row_spec(128), row_spec(1),
            row_spec(NF), row_spec(NC),
            full_spec(kpat), full_spec(expand), full_spec(patt),
            full_spec(Wcat), full_spec(Wu), full_spec(Wd), full_spec(Wf),
            full_spec(b1), full_spec(fn_g), full_spec(fn_b),
            full_spec(ln1_g), full_spec(ln1_b),
            full_spec(W2), full_spec(b2), full_spec(ln2_g), full_spec(ln2_b),
            full_spec(W3), full_spec(b3),
        ],
        out_specs=pl.BlockSpec((MLP_BLK, 1), lambda i: (i, 0)),
        out_shape=jax.ShapeDtypeStruct((B, 1), jnp.float32),
    )(uep, uidx, dep, didx, features, bkt, kpat, expand, patt, Wcat,
      Wu, Wd, Wf, b1, fn_g, fn_b, ln1_g, ln1_b, W2, b2, ln2_g, ln2_b,
      W3, b3)
    return out[:, 0]


def kernel(user_idx, diner_idx, features, categorical_bucket_idx,
           user_table, diner_table, cat_tables, fn_g, fn_b, W1, b1,
           ln1_g, ln1_b, W2, b2, ln2_g, ln2_b, W3, b3):
    uidx = user_idx.astype(jnp.int32)
    didx = diner_idx.astype(jnp.int32)

    user_packed = _tc_repack(user_table.T, 16384)
    diner_packed = _tc_repack(diner_table.T, 16384)
    uep, dep = _sc_gather(user_packed, (uidx // PACK).reshape(1, B),
                          diner_packed, (didx // PACK).reshape(1, B))

    W1c = W1[2 * ED + NF:].reshape(NC, CD, H1)
    Wcat = jnp.einsum("cbd,cdh->cbh", cat_tables, W1c,
                      preferred_element_type=jnp.float32)
    Wcat = Wcat.reshape(NCLS, H1).astype(jnp.bfloat16)

    cls = jnp.arange(NCLS, dtype=jnp.int32)
    expand = (cls[None, :] // NB == jnp.arange(NC, dtype=jnp.int32)[:, None])
    expand = expand.astype(jnp.bfloat16)
    patt = (cls % NB).astype(jnp.float32).reshape(1, NCLS)
    bkt = categorical_bucket_idx.astype(jnp.bfloat16)
    kpat = (jnp.arange(128, dtype=jnp.int32) // ED).reshape(1, 128)

    Wb = W1.astype(jnp.bfloat16)
    Wu = jnp.tile(Wb[:ED], (PACK, 1))
    Wd = jnp.tile(Wb[ED:2 * ED], (PACK, 1))
    Wf = Wb[2 * ED:2 * ED + NF]

    out = _tc_mlp(uep, uidx.reshape(B, 1), dep, didx.reshape(B, 1),
                  features, bkt, kpat, expand, patt, Wcat, Wu, Wd, Wf,
                  b1.reshape(1, H1),
                  fn_g.reshape(1, NF), fn_b.reshape(1, NF),
                  ln1_g.reshape(1, H1), ln1_b.reshape(1, H1),
                  W2.astype(jnp.bfloat16), b2.reshape(1, H2),
                  ln2_g.reshape(1, H2), ln2_b.reshape(1, H2),
                  W3.astype(jnp.bfloat16), b3.reshape(1, 1))
    return out


# speedup vs baseline: 15.8854x; 1.2882x over previous
import dataclasses
import functools

import jax
import jax.numpy as jnp
from jax.experimental import pallas as pl
from jax.experimental.pallas import tpu as pltpu
from jax.experimental.pallas import tpu_sc as plsc

B = 16384
ED = 16
NF = 13
NC, NB, CD = 26, 20, 8
NCLS = NC * NB
H1, H2 = 256, 128
PACK = 128 // ED

GATHER_W = 128
MLP_BLK = 512


REPACK_W = 1024


def _sc_repack(user_tT, diner_tT):
    mesh = plsc.VectorSubcoreMesh(core_axis_name="c", subcore_axis_name="s")
    nu = user_tT.shape[1]
    nd = diner_tT.shape[1]
    cp = pltpu.CompilerParams()
    if "needs_layout_passes" in pltpu.CompilerParams.__dataclass_fields__:
        cp = dataclasses.replace(cp, needs_layout_passes=False)

    @functools.partial(
        pl.kernel,
        out_type=(
            jax.ShapeDtypeStruct((nu // PACK, 128), jnp.float32),
            jax.ShapeDtypeStruct((nd // PACK, 128), jnp.float32),
        ),
        mesh=mesh,
        compiler_params=cp,
    )
    def repack_kernel(ut_hbm, dt_hbm, up_hbm, dp_hbm):
        def body(in_vmem, out_vmem):
            d_vec = jax.lax.iota(jnp.int32, ED)

            @plsc.parallel_loop(0, REPACK_W // PACK)
            def _(p):
                base = jnp.full((ED,), p * PACK, jnp.int32)
                row = jnp.full((ED,), p, jnp.int32)
                for j in range(PACK):
                    v = plsc.load_gather(in_vmem, [d_vec, base + j])
                    plsc.store_scatter(out_vmem, [row, d_vec + j * ED], v)

        def run(t_hbm, out_hbm, n):
            pltpu.emit_pipeline(
                body,
                grid=(n // REPACK_W,),
                in_specs=[pl.BlockSpec((ED, REPACK_W), lambda i: (0, i))],
                out_specs=[pl.BlockSpec((REPACK_W // PACK, 128),
                                        lambda i: (i, 0))],
                core_axis_name=("c", "s"),
                dimension_semantics=(pltpu.PARALLEL,),
            )(t_hbm, out_hbm)

        run(ut_hbm, up_hbm, nu)
        run(dt_hbm, dp_hbm, nd)

    return repack_kernel(user_tT, diner_tT)


def _tail_body(t_ref, packed_ref, out_ref):
    del packed_ref
    x = t_ref[...]
    y = jnp.transpose(x)
    y3 = y.reshape(-1, PACK, ED)
    out_ref[...] = jnp.concatenate([y3[:, j, :] for j in range(PACK)],
                                   axis=1)


def _tc_tail_repack(tT, packed):
    n = tT.shape[1]
    k = n // REPACK_W
    rows = packed.shape[0]
    return pl.pallas_call(
        _tail_body,
        grid=(1,),
        in_specs=[
            pl.BlockSpec((ED, REPACK_W), lambda i: (0, k)),
            pl.BlockSpec(memory_space=pltpu.MemorySpace.HBM),
        ],
        out_specs=pl.BlockSpec((REPACK_W // PACK, 128), lambda i: (k, 0)),
        out_shape=jax.ShapeDtypeStruct((rows, 128), jnp.float32),
        input_output_aliases={1: 0},
    )(tT, packed)


def _sc_gather(user_table, uidx, diner_table, didx):
    mesh = plsc.VectorSubcoreMesh(core_axis_name="c", subcore_axis_name="s")

    @functools.partial(
        pl.kernel,
        out_type=(
            jax.ShapeDtypeStruct((B, 128), jnp.float32),
            jax.ShapeDtypeStruct((B, 128), jnp.float32),
        ),
        mesh=mesh,
    )
    def gather_kernel(ut_hbm, ui_hbm, dt_hbm, di_hbm, ue_hbm, de_hbm):
        def make_body(table_hbm):
            def body(i_vmem, o_vmem):
                pltpu.sync_copy(table_hbm.at[i_vmem.at[0]], o_vmem)
            return body

        def run(table_hbm, idx_hbm, out_hbm):
            pltpu.emit_pipeline(
                make_body(table_hbm),
                grid=(B // GATHER_W,),
                in_specs=[pl.BlockSpec((1, GATHER_W), lambda i: (0, i))],
                out_specs=[pl.BlockSpec((GATHER_W, 128), lambda i: (i, 0))],
                core_axis_name=("c", "s"),
                dimension_semantics=(pltpu.PARALLEL,),
            )(idx_hbm, out_hbm)

        run(ut_hbm, ui_hbm, ue_hbm)
        run(dt_hbm, di_hbm, de_hbm)

    return gather_kernel(user_table, uidx, diner_table, didx)


def _mlp_body(uep, uidx, dep, didx, f, bkt, kpat, expand, patt, Wcat,
              Wu, Wd, Wf, b1, fn_g, fn_b, g1, bb1, W2, b2, g2, bb2,
              W3, b3, out):
    mu = ((uidx[...] % PACK) == kpat[...]).astype(jnp.bfloat16)
    md = ((didx[...] % PACK) == kpat[...]).astype(jnp.bfloat16)
    pu = uep[...].astype(jnp.bfloat16) * mu
    pd = dep[...].astype(jnp.bfloat16) * md

    fx = f[...]
    m = jnp.mean(fx, axis=-1, keepdims=True)
    v = jnp.mean((fx - m) ** 2, axis=-1, keepdims=True)
    fln = (fx - m) * jax.lax.rsqrt(v + 1e-5) * fn_g[...] + fn_b[...]

    rep = jnp.dot(bkt[...], expand[...], preferred_element_type=jnp.float32)
    mh = (rep == patt[...]).astype(jnp.bfloat16)

    h = jnp.dot(mh, Wcat[...], preferred_element_type=jnp.float32)
    h = h + jnp.dot(pu, Wu[...], preferred_element_type=jnp.float32)
    h = h + jnp.dot(pd, Wd[...], preferred_element_type=jnp.float32)
    h = h + jnp.dot(fln.astype(jnp.bfloat16), Wf[...],
                    preferred_element_type=jnp.float32)
    h = h + b1[...]
    m = jnp.mean(h, axis=-1, keepdims=True)
    v = jnp.mean((h - m) ** 2, axis=-1, keepdims=True)
    h = (h - m) * jax.lax.rsqrt(v + 1e-5) * g1[...] + bb1[...]
    h = jnp.maximum(h, 0.0).astype(jnp.bfloat16)

    h = jnp.dot(h, W2[...], preferred_element_type=jnp.float32) + b2[...]
    m = jnp.mean(h, axis=-1, keepdims=True)
    v = jnp.mean((h - m) ** 2, axis=-1, keepdims=True)
    h = (h - m) * jax.lax.rsqrt(v + 1e-5) * g2[...] + bb2[...]
    h = jnp.maximum(h, 0.0).astype(jnp.bfloat16)

    o = jnp.dot(h, W3[...], preferred_element_type=jnp.float32) + b3[...]
    out[...] = jax.nn.sigmoid(o)


def _tc_mlp(uep, uidx, dep, didx, features, bkt, kpat, expand, patt, Wcat,
            Wu, Wd, Wf, b1, fn_g, fn_b, ln1_g, ln1_b, W2, b2, ln2_g, ln2_b,
            W3, b3):
    grid = (B // MLP_BLK,)

    def row_spec(cols):
        return pl.BlockSpec((MLP_BLK, cols), lambda i: (i, 0))

    def full_spec(a):
        return pl.BlockSpec(a.shape, lambda i: (0,) * a.ndim)

    out = pl.pallas_call(
        _mlp_body,
        grid=grid,
        in_specs=[
            row_spec(128), row_spec(1), row_spec(128), row_spec(1),
            row_spec(NF), row_spec(NC),
            full_spec(kpat), full_spec(expand), full_spec(patt),
            full_spec(Wcat), full_spec(Wu), full_spec(Wd), full_spec(Wf),
            full_spec(b1), full_spec(fn_g), full_spec(fn_b),
            full_spec(ln1_g), full_spec(ln1_b),
            full_spec(W2), full_spec(b2), full_spec(ln2_g), full_spec(ln2_b),
            full_spec(W3), full_spec(b3),
        ],
        out_specs=pl.BlockSpec((MLP_BLK, 1), lambda i: (i, 0)),
        out_shape=jax.ShapeDtypeStruct((B, 1), jnp.float32),
    )(uep, uidx, dep, didx, features, bkt, kpat, expand, patt, Wcat,
      Wu, Wd, Wf, b1, fn_g, fn_b, ln1_g, ln1_b, W2, b2, ln2_g, ln2_b,
      W3, b3)
    return out[:, 0]


def kernel(user_idx, diner_idx, features, categorical_bucket_idx,
           user_table, diner_table, cat_tables, fn_g, fn_b, W1, b1,
           ln1_g, ln1_b, W2, b2, ln2_g, ln2_b, W3, b3):
    uidx = user_idx.astype(jnp.int32)
    didx = diner_idx.astype(jnp.int32)

    user_packed, diner_packed = _sc_repack(user_table.T, diner_table.T)
    user_packed = _tc_tail_repack(user_table.T, user_packed)
    diner_packed = _tc_tail_repack(diner_table.T, diner_packed)
    uep, dep = _sc_gather(user_packed, (uidx // PACK).reshape(1, B),
                          diner_packed, (didx // PACK).reshape(1, B))

    W1c = W1[2 * ED + NF:].reshape(NC, CD, H1)
    Wcat = jnp.einsum("cbd,cdh->cbh", cat_tables, W1c,
                      preferred_element_type=jnp.float32)
    Wcat = Wcat.reshape(NCLS, H1).astype(jnp.bfloat16)

    cls = jnp.arange(NCLS, dtype=jnp.int32)
    expand = (cls[None, :] // NB == jnp.arange(NC, dtype=jnp.int32)[:, None])
    expand = expand.astype(jnp.bfloat16)
    patt = (cls % NB).astype(jnp.float32).reshape(1, NCLS)
    bkt = categorical_bucket_idx.astype(jnp.bfloat16)
    kpat = (jnp.arange(128, dtype=jnp.int32) // ED).reshape(1, 128)

    Wb = W1.astype(jnp.bfloat16)
    Wu = jnp.tile(Wb[:ED], (PACK, 1))
    Wd = jnp.tile(Wb[ED:2 * ED], (PACK, 1))
    Wf = Wb[2 * ED:2 * ED + NF]

    out = _tc_mlp(uep, uidx.reshape(B, 1), dep, didx.reshape(B, 1),
                  features, bkt, kpat, expand, patt, Wcat, Wu, Wd, Wf,
                  b1.reshape(1, H1),
                  fn_g.reshape(1, NF), fn_b.reshape(1, NF),
                  ln1_g.reshape(1, H1), ln1_b.reshape(1, H1),
                  W2.astype(jnp.bfloat16), b2.reshape(1, H2),
                  ln2_g.reshape(1, H2), ln2_b.reshape(1, H2),
                  W3.astype(jnp.bfloat16), b3.reshape(1, 1))
    return out
